# baseline (speedup 1.0000x reference)
import functools

import jax
import jax.numpy as jnp
from jax.experimental import pallas as pl
from jax.experimental.pallas import tpu as pltpu

V = 25
NS = 8
HD = 48
KSEL = 9
EPS = 1e-05
BNS = 1e-06 / (1.0 + EPS) ** 0.5
SBN = 1.0 / (1.0 + EPS) ** 0.5


def _shift_edge(a, s):
    if s == 0:
        return a
    n = a.shape[0]
    if s > 0:
        return jnp.concatenate([a[V * s:, :]] + [a[n - V:, :]] * s, axis=0)
    return jnp.concatenate([a[:V, :]] * (-s) + [a[:n + V * s, :]], axis=0)


def _fused_kernel(x_ref, rmean_ref, alearn_ref, alpha_ref, conf_ref,
                  wqbd_ref, bq_ref, wkbd_ref, bk_ref,
                  ww1bd_ref, bw1_ref, ww2_ref, bw2_ref,
                  wdt_ref, bd_ref, r16_ref, c16_ref, m16_ref,
                  wcomb_ref, bcomb_ref, wt1_ref, bb1t_ref,
                  wt2_ref, bb2t_ref,
                  o_ref):
    nb = x_ref.shape[0]
    pre = [_stage_pre(x_ref, rmean_ref, wqbd_ref, bq_ref, wkbd_ref, bk_ref,
                      ww1bd_ref, bw1_ref, ww2_ref, bw2_ref,
                      wdt_ref, bd_ref, conf_ref, s) for s in range(nb)]
    adj = [_stage_adj(pre[s][2], pre[s][3], alearn_ref, alpha_ref,
                      r16_ref, c16_ref, m16_ref) for s in range(nb)]
    for s in range(nb):
        _stage_out(pre[s][0], pre[s][1], adj[s],
                   wcomb_ref, bcomb_ref, wt1_ref, bb1t_ref,
                   wt2_ref, bb2t_ref, o_ref, s)


def _stage_pre(x_ref, rmean_ref, wqbd_ref, bq_ref, wkbd_ref, bk_ref,
               ww1bd_ref, bw1_ref, ww2_ref, bw2_ref,
               wdt_ref, bd_ref, conf_ref, s):
    f32 = jnp.float32

    xf = x_ref[s].T

    t_x = jnp.dot(rmean_ref[...], xf, preferred_element_type=f32)

    bf16 = jnp.bfloat16
    xb = xf.astype(bf16)
    d = jnp.dot(xb, wdt_ref[...], preferred_element_type=f32) + bd_ref[...]
    db = d.astype(bf16)

    q = jnp.dot(t_x, wqbd_ref[...], preferred_element_type=f32) + bq_ref[...]
    k = jnp.dot(t_x, wkbd_ref[...], preferred_element_type=f32) + bk_ref[...]

    h = jnp.dot(t_x, ww1bd_ref[...], preferred_element_type=f32) + bw1_ref[...]
    h = jnp.where(h >= 0, h, 0.01 * h)
    w = jnp.tanh(jax.lax.dot_general(
        h, ww2_ref[...], (((1,), (1,)), ((), ())),
        preferred_element_type=f32) + bw2_ref[...])
    w_raw = jnp.mean(w, axis=0, keepdims=True)
    gl = conf_ref[...] + w_raw
    gl = gl - jnp.max(gl, axis=1, keepdims=True)
    ge = jnp.exp(gl)
    om = ge / jnp.sum(ge, axis=1, keepdims=True)

    ah_parts = []
    for g in range(NS):
        qg = q[:, g * HD:(g + 1) * HD]
        kg = k[:, g * HD:(g + 1) * HD]
        ah_parts.append(jax.lax.dot_general(
            qg, kg, (((1,), (1,)), ((), ())), preferred_element_type=f32))
    ah = jnp.concatenate(ah_parts, axis=0) * (HD ** -0.5)
    return xf, db, ah, om


def _stage_adj(ah, om, alearn_ref, alpha_ref, r16_ref, c16_ref, m16_ref):
    f32 = jnp.float32
    bf16 = jnp.bfloat16
    rows = NS * V
    idxf = jax.lax.broadcasted_iota(jnp.int32, (rows, V), 1).astype(f32)
    cur = ah
    sel = jnp.zeros((rows, V), jnp.bool_)
    for _ in range(KSEL):
        mx = jnp.max(cur, axis=1, keepdims=True)
        cand = cur == mx
        pick_i = jnp.min(jnp.where(cand, idxf, f32(V)), axis=1, keepdims=True)
        pick = idxf == pick_i
        sel = jnp.logical_or(sel, pick)
        cur = jnp.where(pick, -jnp.inf, cur)

    hm = jnp.where(sel, ah, f32(-1e30))
    m = jnp.max(hm, axis=1, keepdims=True)
    e = jnp.exp(hm - m)
    hs = jnp.where(sel, e / jnp.sum(e, axis=1, keepdims=True), f32(0.0))

    a_sem = jnp.zeros((V, V), f32)
    for g in range(NS):
        a_sem = a_sem + om[0:1, g:g + 1] * hs[g * V:(g + 1) * V, :]
    a_sem = a_sem / (jnp.sum(jnp.abs(a_sem), axis=1, keepdims=True) + 1e-08)
    a_fused = alearn_ref[...] + jnp.maximum(alpha_ref[0, 0], 0.0) * a_sem
    a_fused = a_fused * BNS

    ac = jnp.dot(a_fused, c16_ref[...], preferred_element_type=f32)
    a_big = jnp.dot(r16_ref[...], ac, preferred_element_type=f32)
    return a_big.astype(bf16) * m16_ref[...]


def _stage_out(xf, db, a_bigb,
               wcomb_ref, bcomb_ref, wt1_ref, bb1t_ref,
               wt2_ref, bb2t_ref, o_ref, s):
    VT = xf.shape[0]
    C = xf.shape[1]
    T = VT // V
    BC = C // 4
    f32 = jnp.float32
    bf16 = jnp.bfloat16

    kv = 16 * V
    ych = []
    for tc in range(T // 16):
        chunk = db[tc * kv:(tc + 1) * kv, :]
        ych.append(jnp.dot(a_bigb, chunk, preferred_element_type=f32))
    y3 = jnp.concatenate(ych, axis=0)
    y3 = jnp.maximum(y3 + xf, 0.0)
    yb = y3.astype(bf16)

    p_all = (jnp.dot(yb, wcomb_ref[...], preferred_element_type=f32)
             + bcomb_ref[...]) * SBN
    lane = jax.lax.broadcasted_iota(jnp.int32, (VT, C), 1)
    p_all = jnp.where(lane < 3 * BC, jnp.maximum(p_all, 0.0), p_all)

    pball = p_all.astype(bf16)

    p1 = pball[:, 0:BC]
    acc1 = jnp.broadcast_to(bb1t_ref[...], (VT, BC))
    for kk in range(5):
        sh = _shift_edge(p1, (kk - 2) * 1)
        acc1 = acc1 + jax.lax.dot_general(
            sh, wt1_ref[kk], (((1,), (1,)), ((), ())),
            preferred_element_type=f32)
    b1 = acc1 * SBN

    p2 = pball[:, BC:2 * BC]
    acc2 = jnp.broadcast_to(bb2t_ref[...], (VT, BC))
    for kk in range(5):
        sh = _shift_edge(p2, (kk - 2) * 2)
        acc2 = acc2 + jax.lax.dot_general(
            sh, wt2_ref[kk], (((1,), (1,)), ((), ())),
            preferred_element_type=f32)
    b2 = acc2 * SBN

    ninf = jnp.full((V, C), -jnp.inf, f32)
    left = jnp.concatenate([ninf, p_all[:VT - V, :]], axis=0)
    right = jnp.concatenate([p_all[V:, :], ninf], axis=0)
    pooled = jnp.maximum(jnp.maximum(left, p_all), right) * SBN

    out = jnp.concatenate(
        [b1, b2, pooled[:, 2 * BC:3 * BC], p_all[:, 3 * BC:]], axis=1)
    out = jnp.maximum(out + xf, 0.0)
    o_ref[s] = out.T


def kernel(x, PA, edge_importance, alpha, conf_gate, Wq, bq, Wk, bk,
           Ww1, bw1, Ww2, bw2, Wd, bd, Wb1a, bb1a, Wb1t, bb1t,
           Wb2a, bb2a, Wb2t, bb2t, Wb3, bb3, Wb4, bb4):
    N, C, T, Vv = x.shape
    f32 = jnp.float32

    xt = x.reshape(N, C, T * Vv)

    rmean = jnp.tile(jnp.eye(Vv, dtype=f32), (1, T)) / T

    wq_bd = jax.scipy.linalg.block_diag(*jnp.transpose(Wq, (0, 2, 1)))
    wk_bd = jax.scipy.linalg.block_diag(*jnp.transpose(Wk, (0, 2, 1)))
    ww1_bd = jax.scipy.linalg.block_diag(*jnp.transpose(Ww1, (0, 2, 1)))

    al = edge_importance * PA
    al = al / (jnp.sum(jnp.abs(al), axis=1, keepdims=True) + 1e-08)

    wt1 = jnp.transpose(Wb1t[:, :, :, 0], (2, 0, 1))
    wt2 = jnp.transpose(Wb2t[:, :, :, 0], (2, 0, 1))

    bf16 = jnp.bfloat16
    rows16 = jnp.arange(16 * Vv) % Vv
    r16 = jax.nn.one_hot(rows16, Vv, dtype=f32)
    c16 = jax.nn.one_hot(rows16, Vv, dtype=f32).T
    ii = jnp.arange(16 * Vv) // Vv
    m16 = (ii[:, None] == ii[None, :]).astype(bf16)

    wcomb = jnp.concatenate([Wb1a, Wb2a, Wb3, Wb4], axis=0).T
    bcomb = jnp.concatenate([bb1a, bb2a, bb3, bb4]).reshape(1, -1)

    full = lambda shp: pl.BlockSpec(shp, lambda n: (0,) * len(shp))
    args = (
        xt, rmean, al, alpha.reshape(1, 1), conf_gate.reshape(1, NS),
        wq_bd, bq.reshape(1, -1), wk_bd, bk.reshape(1, -1),
        ww1_bd, bw1.reshape(1, -1), Ww2, bw2.reshape(1, -1),
        Wd.T.astype(bf16), bd.reshape(1, -1), r16, c16, m16,
        wcomb.astype(bf16), bcomb, wt1.astype(bf16), bb1t.reshape(1, -1),
        wt2.astype(bf16), bb2t.reshape(1, -1),
    )
    nb = 4 if N % 4 == 0 else 1
    in_specs = [pl.BlockSpec((nb, C, T * Vv), lambda n: (n, 0, 0))]
    in_specs += [full(a.shape) for a in args[1:]]

    out = pl.pallas_call(
        _fused_kernel,
        grid=(N // nb,),
        in_specs=in_specs,
        out_specs=pl.BlockSpec((nb, C, T * Vv), lambda n: (n, 0, 0)),
        out_shape=jax.ShapeDtypeStruct((N, C, T * Vv), f32),
        compiler_params=pltpu.CompilerParams(
            dimension_semantics=("arbitrary",),
        ),
    )(*args)

    return out.reshape(N, C, T, Vv)


# speedup vs baseline: 2.3063x; 2.3063x over previous
import functools

import jax
import jax.numpy as jnp
from jax.experimental import pallas as pl
from jax.experimental.pallas import tpu as pltpu

V = 25
NS = 8
HD = 48
KSEL = 9
EPS = 1e-05
BNS = 1e-06 / (1.0 + EPS) ** 0.5
SBN = 1.0 / (1.0 + EPS) ** 0.5


def _shift_edge(a, s, T):
    if s == 0:
        return a
    if s > 0:
        last = jnp.broadcast_to(a[:, T - 1:T, :], (a.shape[0], s, a.shape[2]))
        return jnp.concatenate([a[:, s:, :], last], axis=1)
    first = jnp.broadcast_to(a[:, 0:1, :], (a.shape[0], -s, a.shape[2]))
    return jnp.concatenate([first, a[:, :T + s, :]], axis=1)


def _fused_kernel(x_ref, alearn_ref, alpha_ref, conf_ref,
                  wqbd_ref, bq_ref, wkbd_ref, bk_ref,
                  ww1bd_ref, bw1_ref, ww2_ref, bw2_ref,
                  wdt_ref, bd_ref, r16_ref, c16_ref, m16_ref,
                  wcomb_ref, bcomb_ref, wt1_ref, bb1t_ref,
                  wt2_ref, bb2t_ref,
                  o_ref):
    nb = x_ref.shape[0]
    pre = [_stage_pre(x_ref, wqbd_ref, bq_ref, wkbd_ref, bk_ref,
                      ww1bd_ref, bw1_ref, ww2_ref, bw2_ref,
                      wdt_ref, bd_ref, conf_ref, s) for s in range(nb)]
    adj = [_stage_adj(pre[s][3], pre[s][4], alearn_ref, alpha_ref,
                      r16_ref, c16_ref, m16_ref) for s in range(nb)]
    for s in range(nb):
        _stage_out(pre[s][0], pre[s][1], pre[s][2], adj[s],
                   wcomb_ref, bcomb_ref, wt1_ref, bb1t_ref,
                   wt2_ref, bb2t_ref, o_ref, s)


def _stage_pre(x_ref, wqbd_ref, bq_ref, wkbd_ref, bk_ref,
               ww1bd_ref, bw1_ref, ww2_ref, bw2_ref,
               wdt_ref, bd_ref, conf_ref, s):
    T = x_ref.shape[2]
    C = x_ref.shape[3]
    VT = V * T
    f32 = jnp.float32

    xv = x_ref[s]
    xf = xv.reshape(VT, C)

    t_x = jnp.mean(xv, axis=1)

    bf16 = jnp.bfloat16
    xb = xf.astype(bf16)
    d = jnp.dot(xb, wdt_ref[...], preferred_element_type=f32) + bd_ref[...]
    db = d.astype(bf16).reshape(V, T, C)

    q = jnp.dot(t_x, wqbd_ref[...], preferred_element_type=f32) + bq_ref[...]
    k = jnp.dot(t_x, wkbd_ref[...], preferred_element_type=f32) + bk_ref[...]

    h = jnp.dot(t_x, ww1bd_ref[...], preferred_element_type=f32) + bw1_ref[...]
    h = jnp.where(h >= 0, h, 0.01 * h)
    w = jnp.tanh(jax.lax.dot_general(
        h, ww2_ref[...], (((1,), (1,)), ((), ())),
        preferred_element_type=f32) + bw2_ref[...])
    w_raw = jnp.mean(w, axis=0, keepdims=True)
    gl = conf_ref[...] + w_raw
    gl = gl - jnp.max(gl, axis=1, keepdims=True)
    ge = jnp.exp(gl)
    om = ge / jnp.sum(ge, axis=1, keepdims=True)

    ah_parts = []
    for g in range(NS):
        qg = q[:, g * HD:(g + 1) * HD]
        kg = k[:, g * HD:(g + 1) * HD]
        ah_parts.append(jax.lax.dot_general(
            qg, kg, (((1,), (1,)), ((), ())), preferred_element_type=f32))
    ah = jnp.concatenate(ah_parts, axis=0) * (HD ** -0.5)
    return xv, xf, db, ah, om


def _stage_adj(ah, om, alearn_ref, alpha_ref, r16_ref, c16_ref, m16_ref):
    f32 = jnp.float32
    bf16 = jnp.bfloat16
    rows = NS * V
    idxf = jax.lax.broadcasted_iota(jnp.int32, (rows, V), 1).astype(f32)
    cur = ah
    sel = jnp.zeros((rows, V), jnp.bool_)
    m = None
    se = None
    for it in range(KSEL):
        mx = jnp.max(cur, axis=1, keepdims=True)
        if it == 0:
            m = mx
            se = jnp.ones_like(mx)
        else:
            se = se + jnp.exp(mx - m)
        cand = cur == mx
        pick_i = jnp.min(jnp.where(cand, idxf, f32(V)), axis=1, keepdims=True)
        pick = idxf == pick_i
        sel = jnp.logical_or(sel, pick)
        cur = jnp.where(pick, -jnp.inf, cur)

    e = jnp.exp(ah - m)
    hs = jnp.where(sel, e / se, f32(0.0))

    a_sem = jnp.zeros((V, V), f32)
    for g in range(NS):
        a_sem = a_sem + om[0:1, g:g + 1] * hs[g * V:(g + 1) * V, :]
    a_fused = alearn_ref[...] + (
        jnp.maximum(alpha_ref[0, 0], 0.0) * (BNS / (1.0 + 1e-08))) * a_sem

    ac = jnp.dot(a_fused, c16_ref[...], preferred_element_type=f32)
    a_big = jnp.dot(r16_ref[...], ac, preferred_element_type=f32)
    return a_big.astype(bf16) * m16_ref[...]


def _stage_out(xv, xf, db, a_bigb,
               wcomb_ref, bcomb_ref, wt1_ref, bb1t_ref,
               wt2_ref, bb2t_ref, o_ref, s):
    T = xv.shape[1]
    C = xv.shape[2]
    VT = V * T
    BC = C // 4
    f32 = jnp.float32
    bf16 = jnp.bfloat16

    kv = 16 * V
    ych = []
    for tc in range(T // 16):
        chunk = db[:, tc * 16:(tc + 1) * 16, :].reshape(kv, C)
        ych.append(jnp.dot(a_bigb, chunk,
                           preferred_element_type=f32).reshape(V, 16, C))
    y3 = jnp.concatenate(ych, axis=1)
    y3 = jnp.maximum(y3 + xv, 0.0)
    yb = y3.astype(bf16).reshape(VT, C)

    p_all = (jnp.dot(yb, wcomb_ref[...], preferred_element_type=f32)
             + bcomb_ref[...]) * SBN
    lane = jax.lax.broadcasted_iota(jnp.int32, (VT, C), 1)
    p_all = jnp.where(lane < 3 * BC, jnp.maximum(p_all, 0.0), p_all)

    pball = p_all.astype(bf16)

    p1 = pball[:, 0:BC].reshape(V, T, BC)
    acc1 = jnp.broadcast_to(bb1t_ref[...], (VT, BC))
    for kk in range(5):
        sh = _shift_edge(p1, (kk - 2) * 1, T).reshape(VT, BC)
        acc1 = acc1 + jax.lax.dot_general(
            sh, wt1_ref[kk], (((1,), (1,)), ((), ())),
            preferred_element_type=f32)
    b1 = acc1 * SBN

    p2 = p_all[:, BC:2 * BC].astype(bf16).reshape(V, T, BC)
    acc2 = jnp.broadcast_to(bb2t_ref[...], (VT, BC))
    for kk in range(5):
        sh = _shift_edge(p2, (kk - 2) * 2, T).reshape(VT, BC)
        acc2 = acc2 + jax.lax.dot_general(
            sh, wt2_ref[kk], (((1,), (1,)), ((), ())),
            preferred_element_type=f32)
    b2 = acc2 * SBN

    p3d = p_all.reshape(V, T, C)
    ninf = jnp.full((V, 1, C), -jnp.inf, f32)
    left = jnp.concatenate([ninf, p3d[:, :T - 1, :]], axis=1)
    right = jnp.concatenate([p3d[:, 1:, :], ninf], axis=1)
    pooled = (jnp.maximum(jnp.maximum(left, p3d), right) * SBN).reshape(VT, C)

    out = jnp.concatenate(
        [b1, b2, pooled[:, 2 * BC:3 * BC], p_all[:, 3 * BC:]], axis=1)
    out = jnp.maximum(out + xf, 0.0)
    o_ref[s] = out.reshape(V, T, C)


def kernel(x, PA, edge_importance, alpha, conf_gate, Wq, bq, Wk, bk,
           Ww1, bw1, Ww2, bw2, Wd, bd, Wb1a, bb1a, Wb1t, bb1t,
           Wb2a, bb2a, Wb2t, bb2t, Wb3, bb3, Wb4, bb4):
    N, C, T, Vv = x.shape
    f32 = jnp.float32

    xt = jnp.transpose(x, (0, 3, 2, 1))

    wq_bd = jax.scipy.linalg.block_diag(*jnp.transpose(Wq, (0, 2, 1)))
    wk_bd = jax.scipy.linalg.block_diag(*jnp.transpose(Wk, (0, 2, 1)))
    ww1_bd = jax.scipy.linalg.block_diag(*jnp.transpose(Ww1, (0, 2, 1)))

    al = edge_importance * PA
    al = al / (jnp.sum(jnp.abs(al), axis=1, keepdims=True) + 1e-08) * BNS

    wt1 = jnp.transpose(Wb1t[:, :, :, 0], (2, 0, 1))
    wt2 = jnp.transpose(Wb2t[:, :, :, 0], (2, 0, 1))

    bf16 = jnp.bfloat16
    rows16 = jnp.arange(16 * Vv) // 16
    r16 = jax.nn.one_hot(rows16, Vv, dtype=f32)
    c16 = jax.nn.one_hot(rows16, Vv, dtype=f32).T
    ii = jnp.arange(16 * Vv) % 16
    m16 = (ii[:, None] == ii[None, :]).astype(bf16)

    wcomb = jnp.concatenate([Wb1a, Wb2a, Wb3, Wb4], axis=0).T
    bcomb = jnp.concatenate([bb1a, bb2a, bb3, bb4]).reshape(1, -1)

    full = lambda shp: pl.BlockSpec(shp, lambda n: (0,) * len(shp))
    args = (
        xt, al, alpha.reshape(1, 1), conf_gate.reshape(1, NS),
        wq_bd, bq.reshape(1, -1), wk_bd, bk.reshape(1, -1),
        ww1_bd, bw1.reshape(1, -1), Ww2, bw2.reshape(1, -1),
        Wd.T.astype(bf16), bd.reshape(1, -1), r16, c16, m16,
        wcomb.astype(bf16), bcomb, wt1.astype(bf16), bb1t.reshape(1, -1),
        wt2.astype(bf16), bb2t.reshape(1, -1),
    )
    nb = 4 if N % 4 == 0 else 1
    in_specs = [pl.BlockSpec((nb, Vv, T, C), lambda n: (n, 0, 0, 0))]
    in_specs += [full(a.shape) for a in args[1:]]

    out = pl.pallas_call(
        _fused_kernel,
        grid=(N // nb,),
        in_specs=in_specs,
        out_specs=pl.BlockSpec((nb, Vv, T, C), lambda n: (n, 0, 0, 0)),
        out_shape=jax.ShapeDtypeStruct((N, Vv, T, C), f32),
        compiler_params=pltpu.CompilerParams(
            dimension_semantics=("arbitrary",),
        ),
    )(*args)

    return jnp.transpose(out, (0, 3, 2, 1))
